# baseline (device time: 60642 ns/iter reference)
import jax
import jax.numpy as jnp
from jax import lax
from jax.experimental import pallas as pl
from jax.experimental.pallas import tpu as pltpu

N_DEV = 8
SQ = 512
SKV = 2048
D_MODEL = 1024
HQ_PER = 8
DH = 128
SCALE = 0.08838834764831843
ROWS_PER = SQ // N_DEV


def _fused(my_arr, xb, Wq, K, V, Wo):

    def body(s_ref, x_ref, wq_ref, k_ref, v_ref, wo_ref, out_ref,
             stage_ref, rs_recv_ref,
             rs_send_sems, rs_recv_sems, ag_send_sems, ag_recv_sems):
        my = s_ref[0]
        c = pl.program_id(0)
        perm = lax.rem(my + 1 + c, N_DEV)
        row_lo = perm * ROWS_PER

        @pl.when(c == 0)
        def _():
            barrier_sem = pltpu.get_barrier_semaphore()
            for p in range(N_DEV):
                @pl.when(p != my)
                def _():
                    pl.semaphore_signal(
                        barrier_sem, inc=1,
                        device_id=(p,), device_id_type=pl.DeviceIdType.MESH,
                    )
            pl.semaphore_wait(barrier_sem, N_DEV - 1)

        q = jnp.dot(x_ref[...], wq_ref[...],
                    preferred_element_type=jnp.float32) * SCALE
        q = q.astype(jnp.bfloat16)
        o_parts = []
        for h in range(HQ_PER):
            qh = q[:, h * DH:(h + 1) * DH]
            s = lax.dot_general(qh, k_ref[h], (((1,), (1,)), ((), ())),
                                preferred_element_type=jnp.float32)
            e = jnp.exp(s)
            l = jnp.sum(e, axis=-1, keepdims=True)
            oh = jnp.dot(e.astype(jnp.bfloat16), v_ref[h],
                         preferred_element_type=jnp.float32) / l
            o_parts.append(oh.astype(jnp.bfloat16))
        o_c = jnp.concatenate(o_parts, axis=1)
        partial_c = jnp.dot(o_c, wo_ref[...],
                            preferred_element_type=jnp.float32)

        @pl.when(c < N_DEV - 1)
        def _():
            stage_ref[pl.ds(row_lo, ROWS_PER), :] = (
                partial_c.astype(jnp.bfloat16))
            rdma = pltpu.make_async_remote_copy(
                src_ref=stage_ref.at[pl.ds(row_lo, ROWS_PER), :],
                dst_ref=rs_recv_ref.at[my],
                send_sem=rs_send_sems.at[c],
                recv_sem=rs_recv_sems.at[my],
                device_id=(perm,),
                device_id_type=pl.DeviceIdType.MESH,
            )
            rdma.start()

        @pl.when(c == N_DEV - 1)
        def _():
            out_ref[pl.ds(row_lo, ROWS_PER), :] = partial_c
            for s_id in range(N_DEV):
                @pl.when(s_id != my)
                def _():
                    recv = pltpu.make_async_remote_copy(
                        src_ref=stage_ref.at[pl.ds(0, ROWS_PER), :],
                        dst_ref=rs_recv_ref.at[s_id],
                        send_sem=rs_send_sems.at[7],
                        recv_sem=rs_recv_sems.at[s_id],
                        device_id=(s_id,),
                        device_id_type=pl.DeviceIdType.MESH,
                    )
                    recv.wait_recv()
                    out_ref[pl.ds(row_lo, ROWS_PER), :] += (
                        rs_recv_ref[s_id].astype(jnp.float32))

            stage_ref[pl.ds(row_lo, ROWS_PER), :] = (
                out_ref[pl.ds(row_lo, ROWS_PER), :].astype(jnp.bfloat16))
            for p in range(N_DEV):
                @pl.when(p != my)
                def _():
                    ag = pltpu.make_async_remote_copy(
                        src_ref=stage_ref.at[pl.ds(row_lo, ROWS_PER), :],
                        dst_ref=stage_ref.at[pl.ds(row_lo, ROWS_PER), :],
                        send_sem=ag_send_sems.at[p],
                        recv_sem=ag_recv_sems.at[my],
                        device_id=(p,),
                        device_id_type=pl.DeviceIdType.MESH,
                    )
                    ag.start()
            for s_id in range(N_DEV):
                @pl.when(s_id != my)
                def _():
                    agr = pltpu.make_async_remote_copy(
                        src_ref=stage_ref.at[pl.ds(0, ROWS_PER), :],
                        dst_ref=stage_ref.at[
                            pl.ds(s_id * ROWS_PER, ROWS_PER), :],
                        send_sem=ag_send_sems.at[my],
                        recv_sem=ag_recv_sems.at[s_id],
                        device_id=(s_id,),
                        device_id_type=pl.DeviceIdType.MESH,
                    )
                    agr.wait_recv()

            out_ref[...] = stage_ref[...].astype(jnp.float32)

            for j in range(N_DEV - 1):
                d = pltpu.make_async_remote_copy(
                    src_ref=stage_ref.at[pl.ds(0, ROWS_PER), :],
                    dst_ref=rs_recv_ref.at[0],
                    send_sem=rs_send_sems.at[j],
                    recv_sem=rs_recv_sems.at[0],
                    device_id=(0,),
                    device_id_type=pl.DeviceIdType.MESH,
                )
                d.wait_send()
            for p in range(N_DEV):
                @pl.when(p != my)
                def _():
                    d = pltpu.make_async_remote_copy(
                        src_ref=stage_ref.at[pl.ds(0, ROWS_PER), :],
                        dst_ref=rs_recv_ref.at[0],
                        send_sem=ag_send_sems.at[p],
                        recv_sem=rs_recv_sems.at[0],
                        device_id=(0,),
                        device_id_type=pl.DeviceIdType.MESH,
                    )
                    d.wait_send()

    grid_spec = pltpu.PrefetchScalarGridSpec(
        num_scalar_prefetch=1,
        grid=(N_DEV,),
        in_specs=[
            pl.BlockSpec((ROWS_PER, D_MODEL),
                         lambda c, s: (lax.rem(s[0] + 1 + c, N_DEV), 0)),
            pl.BlockSpec((D_MODEL, D_MODEL), lambda c, s: (0, 0)),
            pl.BlockSpec((HQ_PER, SKV, DH), lambda c, s: (0, 0, 0)),
            pl.BlockSpec((HQ_PER, SKV, DH), lambda c, s: (0, 0, 0)),
            pl.BlockSpec((D_MODEL, D_MODEL), lambda c, s: (0, 0)),
        ],
        out_specs=pl.BlockSpec((SQ, D_MODEL), lambda c, s: (0, 0)),
        scratch_shapes=[
            pltpu.VMEM((SQ, D_MODEL), jnp.bfloat16),
            pltpu.VMEM((N_DEV, ROWS_PER, D_MODEL), jnp.bfloat16),
            pltpu.SemaphoreType.DMA((N_DEV,)),
            pltpu.SemaphoreType.DMA((N_DEV,)),
            pltpu.SemaphoreType.DMA((N_DEV,)),
            pltpu.SemaphoreType.DMA((N_DEV,)),
        ],
    )
    return pl.pallas_call(
        body,
        grid_spec=grid_spec,
        out_shape=jax.ShapeDtypeStruct((SQ, D_MODEL), jnp.float32),
        compiler_params=pltpu.CompilerParams(
            collective_id=0, dimension_semantics=("arbitrary",)),
    )(my_arr, xb, Wq, K, V, Wo)


def kernel(x, Wq, Wo, K_ext, V_ext):
    my = lax.axis_index("i")

    xb = x[0].astype(jnp.bfloat16)
    K = lax.dynamic_slice_in_dim(K_ext[0], my * HQ_PER, HQ_PER, axis=1)
    V = lax.dynamic_slice_in_dim(V_ext[0], my * HQ_PER, HQ_PER, axis=1)
    K = K.astype(jnp.bfloat16).transpose(1, 0, 2)
    V = V.astype(jnp.bfloat16).transpose(1, 0, 2)
    my_arr = jnp.reshape(my, (1,)).astype(jnp.int32)

    out = _fused(my_arr, xb, Wq.astype(jnp.bfloat16), K, V,
                 Wo.astype(jnp.bfloat16))
    return out.reshape(1, SQ, D_MODEL)


# device time: 56404 ns/iter; 1.0751x vs baseline; 1.0751x over previous
import jax
import jax.numpy as jnp
from jax import lax
from jax.experimental import pallas as pl
from jax.experimental.pallas import tpu as pltpu

N_DEV = 8
SQ = 512
SKV = 2048
D_MODEL = 1024
HQ_PER = 8
DH = 128
SCALE = 0.08838834764831843


def _attention(head0, xb, Wq, K3, V3):

    def _copies(k_hbm, v_hbm, k_stage, v_stage, ksems, vsems, head, slot):
        kc = pltpu.make_async_copy(
            k_hbm.at[:, head, :], k_stage.at[slot], ksems.at[slot])
        vc = pltpu.make_async_copy(
            v_hbm.at[:, head, :], v_stage.at[slot], vsems.at[slot])
        return kc, vc

    def body(h0_ref, x_ref, wq_ref, k_hbm, v_hbm, o_ref,
             k_stage, v_stage, ksems, vsems):
        h = pl.program_id(0)
        head = h0_ref[0] + h
        slot = lax.rem(h, 2)

        @pl.when(h == 0)
        def _():
            for kc in _copies(k_hbm, v_hbm, k_stage, v_stage,
                              ksems, vsems, head, 0):
                kc.start()
            for kc in _copies(k_hbm, v_hbm, k_stage, v_stage,
                              ksems, vsems, head + 1, 1):
                kc.start()

        q = jnp.dot(x_ref[...], wq_ref[...],
                    preferred_element_type=jnp.float32) * SCALE
        q = q.astype(jnp.bfloat16)

        kwait, vwait = _copies(k_hbm, v_hbm, k_stage, v_stage,
                               ksems, vsems, head, slot)
        kwait.wait()
        k = k_stage[slot].astype(jnp.bfloat16)
        s = lax.dot_general(q, k, (((1,), (1,)), ((), ())),
                            preferred_element_type=jnp.float32)

        @pl.when(h < HQ_PER - 2)
        def _():
            kc, _vc = _copies(k_hbm, v_hbm, k_stage, v_stage,
                              ksems, vsems, head + 2, slot)
            kc.start()

        p = jnp.exp(s)
        l = jnp.sum(p, axis=-1, keepdims=True)

        vwait.wait()
        v = v_stage[slot].astype(jnp.bfloat16)
        o = jnp.dot(p.astype(jnp.bfloat16), v,
                    preferred_element_type=jnp.float32) / l

        @pl.when(h < HQ_PER - 2)
        def _():
            _kc, vc = _copies(k_hbm, v_hbm, k_stage, v_stage,
                              ksems, vsems, head + 2, slot)
            vc.start()

        o_ref[...] = o.astype(jnp.bfloat16)

    grid_spec = pltpu.PrefetchScalarGridSpec(
        num_scalar_prefetch=1,
        grid=(HQ_PER,),
        in_specs=[
            pl.BlockSpec((SQ, D_MODEL), lambda h, s: (0, 0)),
            pl.BlockSpec((D_MODEL, DH), lambda h, s: (0, h)),
            pl.BlockSpec(memory_space=pltpu.MemorySpace.HBM),
            pl.BlockSpec(memory_space=pltpu.MemorySpace.HBM),
        ],
        out_specs=pl.BlockSpec((SQ, DH), lambda h, s: (0, h)),
        scratch_shapes=[
            pltpu.VMEM((2, SKV, DH), jnp.float32),
            pltpu.VMEM((2, SKV, DH), jnp.float32),
            pltpu.SemaphoreType.DMA((2,)),
            pltpu.SemaphoreType.DMA((2,)),
        ],
    )
    return pl.pallas_call(
        body,
        grid_spec=grid_spec,
        out_shape=jax.ShapeDtypeStruct((SQ, HQ_PER * DH), jnp.bfloat16),
    )(head0, xb, Wq, K3, V3)


ROWS_PER = SQ // N_DEV


def _project_allreduce(o, Wo):

    def body(o_ref, wo_ref, out_ref, stage_ref, rs_recv_ref, gather_ref,
             rs_send_sems, rs_recv_sems, ag_send_sems, ag_recv_sems):
        my = lax.axis_index("i")

        barrier_sem = pltpu.get_barrier_semaphore()
        for p in range(N_DEV):
            @pl.when(p != my)
            def _():
                pl.semaphore_signal(
                    barrier_sem, inc=1,
                    device_id=(p,), device_id_type=pl.DeviceIdType.MESH,
                )
        pl.semaphore_wait(barrier_sem, N_DEV - 1)

        out_ref[...] = jnp.dot(o_ref[...], wo_ref[...],
                               preferred_element_type=jnp.float32)
        stage_ref[...] = out_ref[...].astype(jnp.bfloat16)

        for p in range(N_DEV):
            @pl.when(p != my)
            def _():
                rdma = pltpu.make_async_remote_copy(
                    src_ref=stage_ref.at[pl.ds(p * ROWS_PER, ROWS_PER), :],
                    dst_ref=rs_recv_ref.at[my],
                    send_sem=rs_send_sems.at[p],
                    recv_sem=rs_recv_sems.at[my],
                    device_id=(p,),
                    device_id_type=pl.DeviceIdType.MESH,
                )
                rdma.start()

        for s in range(N_DEV):
            @pl.when(s != my)
            def _():
                recv = pltpu.make_async_remote_copy(
                    src_ref=stage_ref.at[pl.ds(0, ROWS_PER), :],
                    dst_ref=rs_recv_ref.at[s],
                    send_sem=rs_send_sems.at[s],
                    recv_sem=rs_recv_sems.at[s],
                    device_id=(s,),
                    device_id_type=pl.DeviceIdType.MESH,
                )
                recv.wait_recv()
                out_ref[pl.ds(my * ROWS_PER, ROWS_PER), :] += (
                    rs_recv_ref[s].astype(jnp.float32))

        stage_ref[pl.ds(my * ROWS_PER, ROWS_PER), :] = (
            out_ref[pl.ds(my * ROWS_PER, ROWS_PER), :].astype(jnp.bfloat16))
        gather_ref[pl.ds(my * ROWS_PER, ROWS_PER), :] = (
            stage_ref[pl.ds(my * ROWS_PER, ROWS_PER), :])
        for p in range(N_DEV):
            @pl.when(p != my)
            def _():
                rdma = pltpu.make_async_remote_copy(
                    src_ref=stage_ref.at[pl.ds(my * ROWS_PER, ROWS_PER), :],
                    dst_ref=gather_ref.at[pl.ds(my * ROWS_PER, ROWS_PER), :],
                    send_sem=ag_send_sems.at[p],
                    recv_sem=ag_recv_sems.at[my],
                    device_id=(p,),
                    device_id_type=pl.DeviceIdType.MESH,
                )
                rdma.start()

        for s in range(N_DEV):
            @pl.when(s != my)
            def _():
                recv = pltpu.make_async_remote_copy(
                    src_ref=stage_ref.at[pl.ds(0, ROWS_PER), :],
                    dst_ref=gather_ref.at[pl.ds(s * ROWS_PER, ROWS_PER), :],
                    send_sem=ag_send_sems.at[s],
                    recv_sem=ag_recv_sems.at[s],
                    device_id=(s,),
                    device_id_type=pl.DeviceIdType.MESH,
                )
                recv.wait_recv()

        out_ref[...] = gather_ref[...].astype(jnp.float32)

        for p in range(N_DEV):
            @pl.when(p != my)
            def _():
                for sems in (rs_send_sems, ag_send_sems):
                    drain = pltpu.make_async_remote_copy(
                        src_ref=stage_ref.at[pl.ds(0, ROWS_PER), :],
                        dst_ref=rs_recv_ref.at[0],
                        send_sem=sems.at[p],
                        recv_sem=rs_recv_sems.at[0],
                        device_id=(p,),
                        device_id_type=pl.DeviceIdType.MESH,
                    )
                    drain.wait_send()

    return pl.pallas_call(
        body,
        out_shape=jax.ShapeDtypeStruct((SQ, D_MODEL), jnp.float32),
        in_specs=[
            pl.BlockSpec(memory_space=pltpu.VMEM),
            pl.BlockSpec(memory_space=pltpu.VMEM),
        ],
        out_specs=pl.BlockSpec(memory_space=pltpu.VMEM),
        scratch_shapes=[
            pltpu.VMEM((SQ, D_MODEL), jnp.bfloat16),
            pltpu.VMEM((N_DEV, ROWS_PER, D_MODEL), jnp.bfloat16),
            pltpu.VMEM((SQ, D_MODEL), jnp.bfloat16),
            pltpu.SemaphoreType.DMA((N_DEV,)),
            pltpu.SemaphoreType.DMA((N_DEV,)),
            pltpu.SemaphoreType.DMA((N_DEV,)),
            pltpu.SemaphoreType.DMA((N_DEV,)),
        ],
        compiler_params=pltpu.CompilerParams(collective_id=0),
    )(o, Wo)


def kernel(x, Wq, Wo, K_ext, V_ext):
    my = lax.axis_index("i")

    xb = x[0].astype(jnp.bfloat16)
    head0 = jnp.reshape(my * HQ_PER, (1,)).astype(jnp.int32)

    o = _attention(head0, xb, Wq.astype(jnp.bfloat16), K_ext[0], V_ext[0])
    out = _project_allreduce(o, Wo.astype(jnp.bfloat16))
    return out.reshape(1, SQ, D_MODEL)


# device time: 54804 ns/iter; 1.1065x vs baseline; 1.0292x over previous
import jax
import jax.numpy as jnp
from jax import lax
from jax.experimental import pallas as pl
from jax.experimental.pallas import tpu as pltpu

N_DEV = 8
SQ = 512
SKV = 2048
D_MODEL = 1024
HQ_PER = 8
DH = 128
SCALE = 0.08838834764831843


def _attention(head0, xb, Wq, K3, V3):

    def _copies(k_hbm, v_hbm, k_stage, v_stage, ksems, vsems, head, slot):
        kc = pltpu.make_async_copy(
            k_hbm.at[:, head, :], k_stage.at[slot], ksems.at[slot])
        vc = pltpu.make_async_copy(
            v_hbm.at[:, head, :], v_stage.at[slot], vsems.at[slot])
        return kc, vc

    def body(h0_ref, x_ref, wq_ref, k_hbm, v_hbm, o_ref,
             xb_ref, k_stage, v_stage, ksems, vsems):
        h = pl.program_id(0)
        head = h0_ref[0] + h
        slot = lax.rem(h, 2)

        @pl.when(h == 0)
        def _():
            for kc in _copies(k_hbm, v_hbm, k_stage, v_stage,
                              ksems, vsems, head, 0):
                kc.start()
            for kc in _copies(k_hbm, v_hbm, k_stage, v_stage,
                              ksems, vsems, head + 1, 1):
                kc.start()
            xb_ref[...] = x_ref[...].astype(jnp.bfloat16)

        q = jnp.dot(xb_ref[...], wq_ref[...].astype(jnp.bfloat16),
                    preferred_element_type=jnp.float32) * SCALE
        q = q.astype(jnp.bfloat16)

        kwait, vwait = _copies(k_hbm, v_hbm, k_stage, v_stage,
                               ksems, vsems, head, slot)
        kwait.wait()
        k = k_stage[slot].astype(jnp.bfloat16)
        s = lax.dot_general(q, k, (((1,), (1,)), ((), ())),
                            preferred_element_type=jnp.float32)

        @pl.when(h < HQ_PER - 2)
        def _():
            kc, _vc = _copies(k_hbm, v_hbm, k_stage, v_stage,
                              ksems, vsems, head + 2, slot)
            kc.start()

        p = jnp.exp(s)
        l = jnp.sum(p, axis=-1, keepdims=True)

        vwait.wait()
        v = v_stage[slot].astype(jnp.bfloat16)
        o = jnp.dot(p.astype(jnp.bfloat16), v,
                    preferred_element_type=jnp.float32) / l

        @pl.when(h < HQ_PER - 2)
        def _():
            _kc, vc = _copies(k_hbm, v_hbm, k_stage, v_stage,
                              ksems, vsems, head + 2, slot)
            vc.start()

        o_ref[...] = o.astype(jnp.bfloat16)

    grid_spec = pltpu.PrefetchScalarGridSpec(
        num_scalar_prefetch=1,
        grid=(HQ_PER,),
        in_specs=[
            pl.BlockSpec((SQ, D_MODEL), lambda h, s: (0, 0)),
            pl.BlockSpec((D_MODEL, DH), lambda h, s: (0, h)),
            pl.BlockSpec(memory_space=pltpu.MemorySpace.HBM),
            pl.BlockSpec(memory_space=pltpu.MemorySpace.HBM),
        ],
        out_specs=pl.BlockSpec((SQ, DH), lambda h, s: (0, h)),
        scratch_shapes=[
            pltpu.VMEM((SQ, D_MODEL), jnp.bfloat16),
            pltpu.VMEM((2, SKV, DH), jnp.float32),
            pltpu.VMEM((2, SKV, DH), jnp.float32),
            pltpu.SemaphoreType.DMA((2,)),
            pltpu.SemaphoreType.DMA((2,)),
        ],
    )
    return pl.pallas_call(
        body,
        grid_spec=grid_spec,
        out_shape=jax.ShapeDtypeStruct((SQ, HQ_PER * DH), jnp.bfloat16),
    )(head0, xb, Wq, K3, V3)


ROWS_PER = SQ // N_DEV


def _project_allreduce(o, Wo):

    def body(o_ref, wo_ref, out_ref, stage_ref, rs_recv_ref, gather_ref,
             rs_send_sems, rs_recv_sems, ag_send_sems, ag_recv_sems):
        my = lax.axis_index("i")

        barrier_sem = pltpu.get_barrier_semaphore()
        for p in range(N_DEV):
            @pl.when(p != my)
            def _():
                pl.semaphore_signal(
                    barrier_sem, inc=1,
                    device_id=(p,), device_id_type=pl.DeviceIdType.MESH,
                )
        pl.semaphore_wait(barrier_sem, N_DEV - 1)

        out_ref[...] = jnp.dot(o_ref[...], wo_ref[...].astype(jnp.bfloat16),
                               preferred_element_type=jnp.float32)
        stage_ref[...] = out_ref[...].astype(jnp.bfloat16)

        for p in range(N_DEV):
            @pl.when(p != my)
            def _():
                rdma = pltpu.make_async_remote_copy(
                    src_ref=stage_ref.at[pl.ds(p * ROWS_PER, ROWS_PER), :],
                    dst_ref=rs_recv_ref.at[my],
                    send_sem=rs_send_sems.at[p],
                    recv_sem=rs_recv_sems.at[my],
                    device_id=(p,),
                    device_id_type=pl.DeviceIdType.MESH,
                )
                rdma.start()

        for s in range(N_DEV):
            @pl.when(s != my)
            def _():
                recv = pltpu.make_async_remote_copy(
                    src_ref=stage_ref.at[pl.ds(0, ROWS_PER), :],
                    dst_ref=rs_recv_ref.at[s],
                    send_sem=rs_send_sems.at[s],
                    recv_sem=rs_recv_sems.at[s],
                    device_id=(s,),
                    device_id_type=pl.DeviceIdType.MESH,
                )
                recv.wait_recv()
                out_ref[pl.ds(my * ROWS_PER, ROWS_PER), :] += (
                    rs_recv_ref[s].astype(jnp.float32))

        stage_ref[pl.ds(my * ROWS_PER, ROWS_PER), :] = (
            out_ref[pl.ds(my * ROWS_PER, ROWS_PER), :].astype(jnp.bfloat16))
        gather_ref[pl.ds(my * ROWS_PER, ROWS_PER), :] = (
            stage_ref[pl.ds(my * ROWS_PER, ROWS_PER), :])
        for p in range(N_DEV):
            @pl.when(p != my)
            def _():
                rdma = pltpu.make_async_remote_copy(
                    src_ref=stage_ref.at[pl.ds(my * ROWS_PER, ROWS_PER), :],
                    dst_ref=gather_ref.at[pl.ds(my * ROWS_PER, ROWS_PER), :],
                    send_sem=ag_send_sems.at[p],
                    recv_sem=ag_recv_sems.at[my],
                    device_id=(p,),
                    device_id_type=pl.DeviceIdType.MESH,
                )
                rdma.start()

        for s in range(N_DEV):
            @pl.when(s != my)
            def _():
                recv = pltpu.make_async_remote_copy(
                    src_ref=stage_ref.at[pl.ds(0, ROWS_PER), :],
                    dst_ref=gather_ref.at[pl.ds(s * ROWS_PER, ROWS_PER), :],
                    send_sem=ag_send_sems.at[s],
                    recv_sem=ag_recv_sems.at[s],
                    device_id=(s,),
                    device_id_type=pl.DeviceIdType.MESH,
                )
                recv.wait_recv()

        out_ref[...] = gather_ref[...].astype(jnp.float32)

        for p in range(N_DEV):
            @pl.when(p != my)
            def _():
                for sems in (rs_send_sems, ag_send_sems):
                    drain = pltpu.make_async_remote_copy(
                        src_ref=stage_ref.at[pl.ds(0, ROWS_PER), :],
                        dst_ref=rs_recv_ref.at[0],
                        send_sem=sems.at[p],
                        recv_sem=rs_recv_sems.at[0],
                        device_id=(p,),
                        device_id_type=pl.DeviceIdType.MESH,
                    )
                    drain.wait_send()

    return pl.pallas_call(
        body,
        out_shape=jax.ShapeDtypeStruct((SQ, D_MODEL), jnp.float32),
        in_specs=[
            pl.BlockSpec(memory_space=pltpu.VMEM),
            pl.BlockSpec(memory_space=pltpu.VMEM),
        ],
        out_specs=pl.BlockSpec(memory_space=pltpu.VMEM),
        scratch_shapes=[
            pltpu.VMEM((SQ, D_MODEL), jnp.bfloat16),
            pltpu.VMEM((N_DEV, ROWS_PER, D_MODEL), jnp.bfloat16),
            pltpu.VMEM((SQ, D_MODEL), jnp.bfloat16),
            pltpu.SemaphoreType.DMA((N_DEV,)),
            pltpu.SemaphoreType.DMA((N_DEV,)),
            pltpu.SemaphoreType.DMA((N_DEV,)),
            pltpu.SemaphoreType.DMA((N_DEV,)),
        ],
        compiler_params=pltpu.CompilerParams(collective_id=0),
    )(o, Wo)


def kernel(x, Wq, Wo, K_ext, V_ext):
    my = lax.axis_index("i")

    head0 = jnp.reshape(my * HQ_PER, (1,)).astype(jnp.int32)

    o = _attention(head0, x[0], Wq, K_ext[0], V_ext[0])
    out = _project_allreduce(o, Wo)
    return out.reshape(1, SQ, D_MODEL)
